# baseline (speedup 1.0000x reference)
import functools

import jax
import jax.numpy as jnp
from jax import lax
from jax.experimental import pallas as pl
from jax.experimental.pallas import tpu as pltpu

_B = 32
_N = 32768
_R = _N // 2
_ROWS = 256
_LANES = 128


def _sort_body(xyzT_ref, dir_ref, idx_ref):
    d0 = dir_ref[0, 0, 0]
    d1 = dir_ref[0, 0, 1]
    d2 = dir_ref[0, 0, 2]
    norm = jnp.sqrt((d0 * d0 + d1 * d1) + d2 * d2) + jnp.float32(1e-12)
    n0 = d0 / norm
    n1 = d1 / norm
    n2 = d2 / norm

    X = xyzT_ref[0].reshape(3, _ROWS, _LANES)
    xb0 = X[0].astype(jnp.bfloat16).astype(jnp.float32)
    xb1 = X[1].astype(jnp.bfloat16).astype(jnp.float32)
    xb2 = X[2].astype(jnp.bfloat16).astype(jnp.float32)
    nb0 = jnp.float32(jnp.bfloat16(n0))
    nb1 = jnp.float32(jnp.bfloat16(n1))
    nb2 = jnp.float32(jnp.bfloat16(n2))
    s = xb0 * nb0 + (xb1 * nb1 + xb2 * nb2)

    b = s.view(jnp.int32)
    m = b >> 31
    K = b ^ (m & jnp.int32(0x7FFFFFFF))

    flat = (lax.broadcasted_iota(jnp.int32, (_ROWS, _LANES), 0) * _LANES
            + lax.broadcasted_iota(jnp.int32, (_ROWS, _LANES), 1))
    I = flat

    n = _ROWS * _LANES
    k = 2
    while k <= n:
        j = k // 2
        while j >= 1:
            if j < _LANES:
                axis, amt = 1, j
            else:
                axis, amt = 0, j // _LANES
            Kp = jnp.roll(K, -amt, axis)
            Km = jnp.roll(K, amt, axis)
            Ip = jnp.roll(I, -amt, axis)
            Im = jnp.roll(I, amt, axis)
            is_lower = (flat & j) == 0
            Kpart = jnp.where(is_lower, Kp, Km)
            Ipart = jnp.where(is_lower, Ip, Im)
            self_first = (K > Kpart) | ((K == Kpart) & (I < Ipart))
            dir_desc = (flat & k) == 0
            keep = self_first == (dir_desc == is_lower)
            K = jnp.where(keep, K, Kpart)
            I = jnp.where(keep, I, Ipart)
            j //= 2
        k *= 2

    idx_ref[0] = I[: _R // _LANES]


def _topk_indices(xyzT, direction):
    return pl.pallas_call(
        _sort_body,
        grid=(_B,),
        in_specs=[
            pl.BlockSpec((1, 3, _N), lambda b: (b, 0, 0)),
            pl.BlockSpec((1, 1, 3), lambda b: (b, 0, 0), memory_space=pltpu.SMEM),
        ],
        out_specs=pl.BlockSpec((1, _R // _LANES, _LANES), lambda b: (b, 0, 0)),
        out_shape=jax.ShapeDtypeStruct((_B, _R // _LANES, _LANES), jnp.int32),
    )(xyzT, direction.reshape(_B, 1, 3))


def kernel(xyz, direction):
    xyzT = jnp.swapaxes(xyz, 1, 2)
    idx = _topk_indices(xyzT, direction).reshape(_B, _R)
    return jnp.take_along_axis(xyz, idx[..., None], axis=1)
